# baseline (speedup 1.0000x reference)
import functools

import jax
import jax.numpy as jnp
from jax.experimental import pallas as pl

TOPK = 8
SEASON = (1, 24)
NEG_INF = float("-inf")


def _normalize_body(t_ref, o_ref):
    x = t_ref[0]
    n2 = jnp.sum(x * x, axis=-1, keepdims=True)
    norm = jnp.sqrt(n2)
    o_ref[0] = x / jnp.maximum(norm, 1e-12)


def _mask_body(xq_ref, xk_ref, o_ref, *, tm, l, topk, season):
    q = xq_ref[0]
    k = xk_ref[0]
    sim = jax.lax.dot_general(
        q, k, (((1,), (1,)), ((), ())), preferred_element_type=jnp.float32)

    nc = l // 128
    col = jax.lax.broadcasted_iota(jnp.int32, (tm, l), 1)
    lane = jax.lax.broadcasted_iota(jnp.int32, (tm, 128), 1)
    chunk0 = jax.lax.broadcasted_iota(jnp.int32, (tm, nc, 128), 1)

    simw = sim
    for _ in range(topk):
        v = simw.reshape(tm, nc, 128)
        c = chunk0
        kk = nc
        while kk > 1:
            h = kk // 2
            av, bv = v[:, :h], v[:, h:kk]
            ac, bc = c[:, :h], c[:, h:kk]
            take = bv > av
            v = jnp.where(take, bv, av)
            c = jnp.where(take, bc, ac)
            kk = h
        colv = v[:, 0]
        colc = c[:, 0]
        m = jnp.max(colv, axis=1, keepdims=True)
        cand = jnp.where(colv == m, colc * 128 + lane, l)
        j0 = jnp.min(cand, axis=1, keepdims=True)
        simw = jnp.where(col == j0, NEG_INF, simw)

    i = pl.program_id(1)
    row = i * tm + jax.lax.broadcasted_iota(jnp.int32, (tm, l), 0)
    diff = row - col
    keep = (simw == NEG_INF) | (diff == 0)
    for d in season:
        keep = keep | (diff == d) | (diff == -d)
    o_ref[0, 0] = jnp.where(keep, 0.0, NEG_INF).astype(jnp.float32)


@jax.jit
def kernel(tokens):
    b, l, d = tokens.shape
    tm = 256

    xn = pl.pallas_call(
        _normalize_body,
        grid=(b, l // tm),
        in_specs=[pl.BlockSpec((1, tm, d), lambda bi, i: (bi, i, 0))],
        out_specs=pl.BlockSpec((1, tm, d), lambda bi, i: (bi, i, 0)),
        out_shape=jax.ShapeDtypeStruct((b, l, d), jnp.float32),
    )(tokens)

    body = functools.partial(_mask_body, tm=tm, l=l, topk=min(TOPK, l),
                             season=SEASON)
    out = pl.pallas_call(
        body,
        grid=(b, l // tm),
        in_specs=[
            pl.BlockSpec((1, tm, d), lambda bi, i: (bi, i, 0)),
            pl.BlockSpec((1, l, d), lambda bi, i: (bi, 0, 0)),
        ],
        out_specs=pl.BlockSpec((1, 1, tm, l), lambda bi, i: (bi, 0, i, 0)),
        out_shape=jax.ShapeDtypeStruct((b, 1, l, l), jnp.float32),
    )(xn, xn)
    return out


# speedup vs baseline: 1.8246x; 1.8246x over previous
import functools

import jax
import jax.numpy as jnp
from jax.experimental import pallas as pl
from jax.experimental.pallas import tpu as pltpu

TOPK = 8
SEASON = (1, 24)
NEG_INF = float("-inf")


def _normalize_body(t_ref, o_ref):
    x = t_ref[0]
    n2 = jnp.sum(x * x, axis=-1, keepdims=True)
    norm = jnp.sqrt(n2)
    o_ref[0] = x / jnp.maximum(norm, 1e-12)


def _mask_body(xq_ref, xk_ref, o_ref, *, tm, l, topk, season):
    q = xq_ref[0]
    k = xk_ref[0]
    sim = jax.lax.dot_general(
        q, k, (((1,), (1,)), ((), ())), preferred_element_type=jnp.float32)

    col = jax.lax.broadcasted_iota(jnp.int32, (tm, l), 1)

    simw = sim
    for _ in range(topk):
        m = jnp.max(simw, axis=1, keepdims=True)
        cand = jnp.where(simw == m, col, l)
        j0 = jnp.min(cand, axis=1, keepdims=True)
        simw = jnp.where(cand == j0, NEG_INF, simw)

    i = pl.program_id(1)
    row = i * tm + jax.lax.broadcasted_iota(jnp.int32, (tm, l), 0)
    diff = row - col
    keep = (simw == NEG_INF) | (diff == 0)
    for d in season:
        keep = keep | (diff == d) | (diff == -d)
    o_ref[0, 0] = jnp.where(keep, 0.0, NEG_INF).astype(jnp.float32)


@jax.jit
def kernel(tokens):
    b, l, d = tokens.shape
    tm = 256

    xn = pl.pallas_call(
        _normalize_body,
        grid=(b, l // tm),
        in_specs=[pl.BlockSpec((1, tm, d), lambda bi, i: (bi, i, 0))],
        out_specs=pl.BlockSpec((1, tm, d), lambda bi, i: (bi, i, 0)),
        out_shape=jax.ShapeDtypeStruct((b, l, d), jnp.float32),
        compiler_params=pltpu.CompilerParams(
            dimension_semantics=("parallel", "parallel")),
    )(tokens)

    body = functools.partial(_mask_body, tm=tm, l=l, topk=min(TOPK, l),
                             season=SEASON)
    out = pl.pallas_call(
        body,
        grid=(b, l // tm),
        in_specs=[
            pl.BlockSpec((1, tm, d), lambda bi, i: (bi, i, 0)),
            pl.BlockSpec((1, l, d), lambda bi, i: (bi, 0, 0)),
        ],
        out_specs=pl.BlockSpec((1, 1, tm, l), lambda bi, i: (bi, 0, i, 0)),
        out_shape=jax.ShapeDtypeStruct((b, 1, l, l), jnp.float32),
        compiler_params=pltpu.CompilerParams(
            dimension_semantics=("parallel", "parallel")),
    )(xn, xn)
    return out


# speedup vs baseline: 2.0591x; 1.1286x over previous
import functools

import jax
import jax.numpy as jnp
from jax.experimental import pallas as pl
from jax.experimental.pallas import tpu as pltpu

TOPK = 8
SEASON = (1, 24)
NEG_INF = float("-inf")


def _normalize_body(t_ref, o_ref):
    x = t_ref[0]
    n2 = jnp.sum(x * x, axis=-1, keepdims=True)
    norm = jnp.sqrt(n2)
    o_ref[0] = x / jnp.maximum(norm, 1e-12)


def _mask_body(xq_ref, xk_ref, o_ref, *, tm, l, topk, season):
    q = xq_ref[0]
    k = xk_ref[0]
    sim = jax.lax.dot_general(
        q, k, (((1,), (1,)), ((), ())), preferred_element_type=jnp.float32)

    col = jax.lax.broadcasted_iota(jnp.int32, (tm, l), 1)

    simw = sim
    for _ in range(topk):
        m = jnp.max(simw, axis=1, keepdims=True)
        cand = jnp.where(simw == m, col, l)
        j0 = jnp.min(cand, axis=1, keepdims=True)
        simw = jnp.where(cand == j0, NEG_INF, simw)

    i = pl.program_id(1)
    row = i * tm + jax.lax.broadcasted_iota(jnp.int32, (tm, l), 0)
    diff = row - col
    keep = (simw == NEG_INF) | (diff == 0)
    for d in season:
        keep = keep | (diff == d) | (diff == -d)
    o_ref[0, 0] = jnp.where(keep, 0.0, NEG_INF).astype(jnp.float32)


@jax.jit
def kernel(tokens):
    b, l, d = tokens.shape
    tm = 512

    xn = pl.pallas_call(
        _normalize_body,
        grid=(b, l // tm),
        in_specs=[pl.BlockSpec((1, tm, d), lambda bi, i: (bi, i, 0))],
        out_specs=pl.BlockSpec((1, tm, d), lambda bi, i: (bi, i, 0)),
        out_shape=jax.ShapeDtypeStruct((b, l, d), jnp.float32),
        compiler_params=pltpu.CompilerParams(
            dimension_semantics=("parallel", "parallel")),
    )(tokens)

    body = functools.partial(_mask_body, tm=tm, l=l, topk=min(TOPK, l),
                             season=SEASON)
    out = pl.pallas_call(
        body,
        grid=(b, l // tm),
        in_specs=[
            pl.BlockSpec((1, tm, d), lambda bi, i: (bi, i, 0)),
            pl.BlockSpec((1, l, d), lambda bi, i: (bi, 0, 0)),
        ],
        out_specs=pl.BlockSpec((1, 1, tm, l), lambda bi, i: (bi, 0, i, 0)),
        out_shape=jax.ShapeDtypeStruct((b, 1, l, l), jnp.float32),
        compiler_params=pltpu.CompilerParams(
            dimension_semantics=("parallel", "parallel")),
    )(xn, xn)
    return out


# speedup vs baseline: 2.0972x; 1.0185x over previous
import functools

import jax
import jax.numpy as jnp
from jax.experimental import pallas as pl
from jax.experimental.pallas import tpu as pltpu

TOPK = 8
SEASON = (1, 24)
NEG_INF = float("-inf")


def _normalize_body(t_ref, o_ref):
    x = t_ref[0]
    n2 = jnp.sum(x * x, axis=-1, keepdims=True)
    norm = jnp.sqrt(n2)
    o_ref[0] = x / jnp.maximum(norm, 1e-12)


def _mask_body(xq_ref, xk_ref, o_ref, *, tm, l, topk, season):
    q = xq_ref[0]
    k = xk_ref[0]
    sim = jax.lax.dot_general(
        q, k, (((1,), (1,)), ((), ())), preferred_element_type=jnp.float32)

    col = jax.lax.broadcasted_iota(jnp.int32, (tm, l), 1)

    simw = sim
    for _ in range(topk):
        m = jnp.max(simw, axis=1, keepdims=True)
        cand = jnp.where(simw == m, col, l)
        j0 = jnp.min(cand, axis=1, keepdims=True)
        simw = jnp.where(cand == j0, NEG_INF, simw)

    i = pl.program_id(1)
    row = i * tm + jax.lax.broadcasted_iota(jnp.int32, (tm, l), 0)
    diff = row - col
    keep = (simw == NEG_INF) | (diff == 0)
    for d in season:
        keep = keep | (diff == d) | (diff == -d)
    o_ref[0, 0] = jnp.where(keep, 0.0, NEG_INF).astype(jnp.float32)


@jax.jit
def kernel(tokens):
    b, l, d = tokens.shape
    tm = 1024

    xn = pl.pallas_call(
        _normalize_body,
        grid=(b, l // tm),
        in_specs=[pl.BlockSpec((1, tm, d), lambda bi, i: (bi, i, 0))],
        out_specs=pl.BlockSpec((1, tm, d), lambda bi, i: (bi, i, 0)),
        out_shape=jax.ShapeDtypeStruct((b, l, d), jnp.float32),
        compiler_params=pltpu.CompilerParams(
            dimension_semantics=("parallel", "parallel")),
    )(tokens)

    body = functools.partial(_mask_body, tm=tm, l=l, topk=min(TOPK, l),
                             season=SEASON)
    out = pl.pallas_call(
        body,
        grid=(b, l // tm),
        in_specs=[
            pl.BlockSpec((1, tm, d), lambda bi, i: (bi, i, 0)),
            pl.BlockSpec((1, l, d), lambda bi, i: (bi, 0, 0)),
        ],
        out_specs=pl.BlockSpec((1, 1, tm, l), lambda bi, i: (bi, 0, i, 0)),
        out_shape=jax.ShapeDtypeStruct((b, 1, l, l), jnp.float32),
        compiler_params=pltpu.CompilerParams(
            dimension_semantics=("parallel", "parallel")),
    )(xn, xn)
    return out


# speedup vs baseline: 2.4050x; 1.1468x over previous
import functools

import jax
import jax.numpy as jnp
from jax.experimental import pallas as pl
from jax.experimental.pallas import tpu as pltpu

TOPK = 8
SEASON = (1, 24)
NEG_INF = float("-inf")


def _normalize_body(t_ref, o_ref):
    x = t_ref[0]
    n2 = jnp.sum(x * x, axis=-1, keepdims=True)
    norm = jnp.sqrt(n2)
    o_ref[0] = x / jnp.maximum(norm, 1e-12)


def _mask_body(xq_ref, xk_ref, band_ref, o_ref, *, tm, l, topk):
    q = xq_ref[0]
    k = xk_ref[0]
    sim = jax.lax.dot_general(
        q, k, (((1,), (1,)), ((), ())), preferred_element_type=jnp.float32)

    colf = jax.lax.broadcasted_iota(jnp.int32, (tm, l), 1).astype(jnp.float32)

    simw = sim
    for _ in range(topk):
        m = jnp.max(simw, axis=1, keepdims=True)
        cand = jnp.where(simw == m, colf, float(l))
        j0 = jnp.min(cand, axis=1, keepdims=True)
        simw = jnp.where(cand == j0, NEG_INF, simw)

    keep = (simw == NEG_INF) | (band_ref[0] != 0)
    o_ref[0, 0] = jnp.where(keep, 0.0, NEG_INF).astype(jnp.float32)


@jax.jit
def kernel(tokens):
    b, l, d = tokens.shape
    tm = 1024

    xn = pl.pallas_call(
        _normalize_body,
        grid=(b, l // tm),
        in_specs=[pl.BlockSpec((1, tm, d), lambda bi, i: (bi, i, 0))],
        out_specs=pl.BlockSpec((1, tm, d), lambda bi, i: (bi, i, 0)),
        out_shape=jax.ShapeDtypeStruct((b, l, d), jnp.float32),
        compiler_params=pltpu.CompilerParams(
            dimension_semantics=("parallel", "parallel")),
    )(tokens)

    ii = jnp.arange(l, dtype=jnp.int32)[:, None]
    jj = jnp.arange(l, dtype=jnp.int32)[None, :]
    ad = jnp.abs(ii - jj)
    band = ad == 0
    for s in SEASON:
        band = band | (ad == s)
    band = band.astype(jnp.int8)[None]

    body = functools.partial(_mask_body, tm=tm, l=l, topk=min(TOPK, l))
    out = pl.pallas_call(
        body,
        grid=(b, l // tm),
        in_specs=[
            pl.BlockSpec((1, tm, d), lambda bi, i: (bi, i, 0)),
            pl.BlockSpec((1, l, d), lambda bi, i: (bi, 0, 0)),
            pl.BlockSpec((1, tm, l), lambda bi, i: (0, i, 0)),
        ],
        out_specs=pl.BlockSpec((1, 1, tm, l), lambda bi, i: (bi, 0, i, 0)),
        out_shape=jax.ShapeDtypeStruct((b, 1, l, l), jnp.float32),
        compiler_params=pltpu.CompilerParams(
            dimension_semantics=("parallel", "parallel")),
    )(xn, xn, band)
    return out
